# Initial kernel scaffold; baseline (speedup 1.0000x reference)
import jax
import jax.numpy as jnp
from jax.experimental import pallas as pl


def kernel(x, edge_index, batch, W1, b1, g1, be1, W2, b2, g2, be2, W3, b3):
    raise NotImplementedError("write your pallas kernel here")


# speedup vs baseline: 14.0985x; 14.0985x over previous
import functools

import jax
import jax.numpy as jnp
from jax import lax
from jax.experimental import pallas as pl
from jax.experimental.pallas import tpu as pltpu
from jax.experimental.pallas import tpu_sc as plsc

N = 10000
E = 320000
F_IN = 128
H1 = 64
H2 = 128
C = 10
G = 64

NP = 10240
BLK = 640
GRID = NP // BLK
NW = 32
KCH = 128
KI = 79
EP = NW * KI * KCH
ROWS_PER_TILE = NP // 16


def _sc_mesh():
    return plsc.VectorSubcoreMesh(core_axis_name="c", subcore_axis_name="s")


def _make_deg_kernel():
    @functools.partial(
        pl.kernel,
        out_type=jax.ShapeDtypeStruct((2, NP, 16), jnp.float32),
        mesh=_sc_mesh(),
        scratch_types=[
            pltpu.VMEM((KI, KCH), jnp.int32),
            pltpu.VMEM((KCH, 16), jnp.float32),
            pltpu.VMEM_SHARED((NP, 16), jnp.float32),
        ],
        compiler_params=pltpu.CompilerParams(use_tc_tiling_on_sc=False),
    )
    def deg_kernel(dst_hbm, ones_hbm, zeros_hbm, out_hbm, dstv, onesv, acc):
        cid = lax.axis_index("c")
        sid = lax.axis_index("s")
        wid = cid * 16 + sid
        pltpu.sync_copy(dst_hbm.at[wid], dstv)
        pltpu.sync_copy(ones_hbm, onesv)
        rs = sid * ROWS_PER_TILE
        pltpu.sync_copy(zeros_hbm.at[pl.ds(rs, ROWS_PER_TILE)],
                        acc.at[pl.ds(rs, ROWS_PER_TILE)])
        plsc.subcore_barrier()

        def body(j, carry):
            pltpu.sync_copy(onesv, acc.at[dstv.at[j]], add=True)
            return carry

        lax.fori_loop(0, KI, body, 0)
        plsc.subcore_barrier()
        pltpu.sync_copy(acc.at[pl.ds(rs, ROWS_PER_TILE)],
                        out_hbm.at[cid, pl.ds(rs, ROWS_PER_TILE)])

    return deg_kernel


def _make_agg_kernel(h):
    @functools.partial(
        pl.kernel,
        out_type=jax.ShapeDtypeStruct((2, NP, h), jnp.float32),
        mesh=_sc_mesh(),
        scratch_types=[
            pltpu.VMEM((KI, KCH), jnp.int32),
            pltpu.VMEM((KI, KCH), jnp.int32),
            pltpu.VMEM((KCH, h), jnp.float32),
            pltpu.VMEM_SHARED((NP, h), jnp.float32),
            pltpu.SemaphoreType.DMA,
        ],
        compiler_params=pltpu.CompilerParams(use_tc_tiling_on_sc=False),
    )
    def agg_kernel(y_hbm, src_hbm, dst_hbm, zeros_hbm, out_hbm,
                   srcv, dstv, rows, acc, sem):
        cid = lax.axis_index("c")
        sid = lax.axis_index("s")
        wid = cid * 16 + sid
        pltpu.sync_copy(src_hbm.at[wid], srcv)
        pltpu.sync_copy(dst_hbm.at[wid], dstv)
        rs = sid * ROWS_PER_TILE
        pltpu.sync_copy(zeros_hbm.at[pl.ds(rs, ROWS_PER_TILE)],
                        acc.at[pl.ds(rs, ROWS_PER_TILE)])
        plsc.subcore_barrier()

        def body(j, carry):
            pltpu.async_copy(y_hbm.at[srcv.at[j]], rows, sem).wait()
            pltpu.sync_copy(rows, acc.at[dstv.at[j]], add=True)
            return carry

        lax.fori_loop(0, KI, body, 0)
        plsc.subcore_barrier()
        pltpu.sync_copy(acc.at[pl.ds(rs, ROWS_PER_TILE)],
                        out_hbm.at[cid, pl.ds(rs, ROWS_PER_TILE)])

    return agg_kernel


def _row_mask(i, rows, cols):
    gid = i * BLK + lax.broadcasted_iota(jnp.int32, (rows, cols), 0)
    return (gid < N).astype(jnp.float32)


def _tc1_body(x_ref, d0_ref, d1_ref, w_ref, y_ref, dis_ref):
    deg = d0_ref[:, 0:1] + d1_ref[:, 0:1] + 1.0
    dis = lax.rsqrt(deg)
    xw = jnp.dot(x_ref[...], w_ref[...], preferred_element_type=jnp.float32)
    y_ref[...] = xw * dis
    dis_ref[...] = jnp.broadcast_to(dis, (BLK, 16))


def _tc1(x_pad, deg0, deg1, W1):
    return pl.pallas_call(
        _tc1_body,
        grid=(GRID,),
        in_specs=[
            pl.BlockSpec((BLK, F_IN), lambda i: (i, 0)),
            pl.BlockSpec((BLK, 16), lambda i: (i, 0)),
            pl.BlockSpec((BLK, 16), lambda i: (i, 0)),
            pl.BlockSpec((F_IN, H1), lambda i: (0, 0)),
        ],
        out_specs=[
            pl.BlockSpec((BLK, H1), lambda i: (i, 0)),
            pl.BlockSpec((BLK, 16), lambda i: (i, 0)),
        ],
        out_shape=[
            jax.ShapeDtypeStruct((NP, H1), jnp.float32),
            jax.ShapeDtypeStruct((NP, 16), jnp.float32),
        ],
    )(x_pad, deg0, deg1, W1)


def _comb_body(h, a0_ref, a1_ref, y_ref, dis_ref, b_ref, z_ref, st_ref):
    i = pl.program_id(0)
    z = (a0_ref[...] + a1_ref[...] + y_ref[...]) * dis_ref[:, 0:1] + b_ref[...]
    z_ref[...] = z
    zm = z * _row_mask(i, BLK, h)

    @pl.when(i == 0)
    def _():
        st_ref[...] = jnp.zeros_like(st_ref)

    st_ref[0:1, :] += jnp.sum(zm, axis=0, keepdims=True)
    st_ref[1:2, :] += jnp.sum(zm * zm, axis=0, keepdims=True)


def _comb(h, a0, a1, y, dis, b):
    return pl.pallas_call(
        functools.partial(_comb_body, h),
        grid=(GRID,),
        in_specs=[
            pl.BlockSpec((BLK, h), lambda i: (i, 0)),
            pl.BlockSpec((BLK, h), lambda i: (i, 0)),
            pl.BlockSpec((BLK, h), lambda i: (i, 0)),
            pl.BlockSpec((BLK, 16), lambda i: (i, 0)),
            pl.BlockSpec((1, h), lambda i: (0, 0)),
        ],
        out_specs=[
            pl.BlockSpec((BLK, h), lambda i: (i, 0)),
            pl.BlockSpec((8, h), lambda i: (0, 0)),
        ],
        out_shape=[
            jax.ShapeDtypeStruct((NP, h), jnp.float32),
            jax.ShapeDtypeStruct((8, h), jnp.float32),
        ],
    )(a0, a1, y, dis, b)


def _bn_mm_body(h, hn, z_ref, st_ref, g_ref, be_ref, dis_ref, w_ref, y_ref):
    i = pl.program_id(0)
    mu = st_ref[0:1, :] / float(N)
    var = st_ref[1:2, :] / float(N) - mu * mu
    hh = (z_ref[...] - mu) * lax.rsqrt(var + 1e-5) * g_ref[...] + be_ref[...]
    hh = jnp.maximum(hh, 0.0)
    yn = jnp.dot(hh, w_ref[...], preferred_element_type=jnp.float32)
    y_ref[...] = yn * dis_ref[:, 0:1] * _row_mask(i, BLK, hn)


def _bn_mm(h, hn, z, st, g, be, dis, w):
    return pl.pallas_call(
        functools.partial(_bn_mm_body, h, hn),
        grid=(GRID,),
        in_specs=[
            pl.BlockSpec((BLK, h), lambda i: (i, 0)),
            pl.BlockSpec((8, h), lambda i: (0, 0)),
            pl.BlockSpec((1, h), lambda i: (0, 0)),
            pl.BlockSpec((1, h), lambda i: (0, 0)),
            pl.BlockSpec((BLK, 16), lambda i: (i, 0)),
            pl.BlockSpec((h, hn), lambda i: (0, 0)),
        ],
        out_specs=pl.BlockSpec((BLK, hn), lambda i: (i, 0)),
        out_shape=jax.ShapeDtypeStruct((NP, hn), jnp.float32),
    )(z, st, g, be, dis, w)


def _final_body(a0_ref, a1_ref, y_ref, dis_ref, b_ref, batch_ref,
                pooled_ref, logp_ref):
    i = pl.program_id(0)
    z = (a0_ref[...] + a1_ref[...] + y_ref[...]) * dis_ref[:, 0:1] + b_ref[...]
    mask = _row_mask(i, BLK, 16)
    colid = lax.broadcasted_iota(jnp.int32, (BLK, 16), 1)
    zaug = (z + (colid == 15).astype(jnp.float32)) * mask
    gid = lax.broadcasted_iota(jnp.int32, (G, BLK), 0)
    oh = (gid == batch_ref[...].reshape(1, BLK)).astype(jnp.float32)
    p = jnp.dot(oh, zaug, preferred_element_type=jnp.float32)

    @pl.when(i == 0)
    def _():
        pooled_ref[...] = jnp.zeros_like(pooled_ref)

    pooled_ref[...] += p

    @pl.when(i == GRID - 1)
    def _():
        pooled = pooled_ref[...]
        cnt = jnp.maximum(pooled[:, 15:16], 1.0)
        mean = pooled / cnt
        cid2 = lax.broadcasted_iota(jnp.int32, (G, 16), 1)
        meff = jnp.where(cid2 < C, mean, -jnp.inf)
        mx = jnp.max(meff, axis=1, keepdims=True)
        ls = jnp.log(jnp.sum(jnp.exp(meff - mx), axis=1, keepdims=True))
        logp_ref[...] = meff - mx - ls


def _final(a0, a1, y, dis, b, batch2d):
    return pl.pallas_call(
        _final_body,
        grid=(GRID,),
        in_specs=[
            pl.BlockSpec((BLK, 16), lambda i: (i, 0)),
            pl.BlockSpec((BLK, 16), lambda i: (i, 0)),
            pl.BlockSpec((BLK, 16), lambda i: (i, 0)),
            pl.BlockSpec((BLK, 16), lambda i: (i, 0)),
            pl.BlockSpec((1, 16), lambda i: (0, 0)),
            pl.BlockSpec((1, 1, BLK), lambda i: (i, 0, 0)),
        ],
        out_specs=[
            pl.BlockSpec((G, 16), lambda i: (0, 0)),
            pl.BlockSpec((G, 16), lambda i: (0, 0)),
        ],
        out_shape=[
            jax.ShapeDtypeStruct((G, 16), jnp.float32),
            jax.ShapeDtypeStruct((G, 16), jnp.float32),
        ],
    )(a0, a1, y, dis, b, batch2d)


def kernel(x, edge_index, batch, W1, b1, g1, be1, W2, b2, g2, be2, W3, b3):
    f32 = jnp.float32
    x_pad = jnp.pad(x, ((0, NP - N), (0, 0)))
    src = jnp.pad(edge_index[0], (0, EP - E), constant_values=N)
    dst = jnp.pad(edge_index[1], (0, EP - E), constant_values=N)
    src3 = src.reshape(NW, KI, KCH)
    dst3 = dst.reshape(NW, KI, KCH)
    batch2d = jnp.pad(batch, (0, NP - N), constant_values=G).reshape(GRID, 1, BLK)
    W3p = jnp.pad(W3, ((0, 0), (0, 16 - C)))
    b3p = jnp.pad(b3, (0, 16 - C)).reshape(1, 16)
    ones16 = jnp.ones((KCH, 16), f32)
    z16 = jnp.zeros((NP, 16), f32)
    z64 = jnp.zeros((NP, H1), f32)
    z128 = jnp.zeros((NP, H2), f32)
    b1r = b1.reshape(1, H1)
    g1r = g1.reshape(1, H1)
    be1r = be1.reshape(1, H1)
    b2r = b2.reshape(1, H2)
    g2r = g2.reshape(1, H2)
    be2r = be2.reshape(1, H2)

    degp = _make_deg_kernel()(dst3, ones16, z16)
    deg0, deg1 = degp[0], degp[1]

    y1, dis = _tc1(x_pad, deg0, deg1, W1)

    ag = _make_agg_kernel(H1)(y1, src3, dst3, z64)
    z1, st1 = _comb(H1, ag[0], ag[1], y1, dis, b1r)
    y2 = _bn_mm(H1, H2, z1, st1, g1r, be1r, dis, W2)

    ag = _make_agg_kernel(H2)(y2, src3, dst3, z128)
    z2, st2 = _comb(H2, ag[0], ag[1], y2, dis, b2r)
    y3 = _bn_mm(H2, 16, z2, st2, g2r, be2r, dis, W3p)

    ag = _make_agg_kernel(16)(y3, src3, dst3, z16)
    _, logp = _final(ag[0], ag[1], y3, dis, b3p, batch2d)
    return logp[:, :C]
